# baseline (speedup 1.0000x reference)
import jax
import jax.numpy as jnp
from jax.experimental import pallas as pl
from jax.experimental.pallas import tpu as pltpu


def _make_dma_kernel(B, S, D):
    def _kern(tid_ref, x_hbm, p_hbm, o_hbm, sem_big, sem_rows):
        big = pltpu.make_async_copy(x_hbm, o_hbm.at[:, pl.ds(0, S), :], sem_big)
        big.start()
        rows = []
        for b in range(B):
            c = pltpu.make_async_copy(
                p_hbm.at[pl.ds(tid_ref[b], 1), :],
                o_hbm.at[b, pl.ds(S, 1), :],
                sem_rows,
            )
            c.start()
            rows.append(c)
        big.wait()
        for c in rows:
            c.wait()

    return _kern


def kernel(x, task_id, prompt):
    B, S, D = x.shape
    task_id32 = task_id.astype(jnp.int32)

    out = pl.pallas_call(
        _make_dma_kernel(B, S, D),
        in_specs=[
            pl.BlockSpec(memory_space=pltpu.MemorySpace.SMEM),
            pl.BlockSpec(memory_space=pltpu.MemorySpace.HBM),
            pl.BlockSpec(memory_space=pltpu.MemorySpace.HBM),
        ],
        out_specs=pl.BlockSpec(memory_space=pltpu.MemorySpace.HBM),
        out_shape=jax.ShapeDtypeStruct((B, S + 1, D), x.dtype),
        scratch_shapes=[pltpu.SemaphoreType.DMA, pltpu.SemaphoreType.DMA],
    )(task_id32, x, prompt)
    return (out, task_id)
